# baseline (speedup 1.0000x reference)
import functools

import jax
import jax.numpy as jnp
from jax import lax
from jax.experimental import pallas as pl
from jax.experimental.pallas import tpu as pltpu
from jax.experimental.pallas import tpu_sc as plsc

B = 16384
F = 26
V = 100000
E = 16
ND = 13
H1, H2 = 256, 128

_NC = 2
_NS = 16
_NW = _NC * _NS
_BPW = B // _NW
_CB = 128
_NCHUNK = _BPW // _CB


def _build_gather():
    mesh = plsc.VectorSubcoreMesh(core_axis_name="c", subcore_axis_name="s")

    @functools.partial(
        pl.kernel,
        mesh=mesh,
        compiler_params=pltpu.CompilerParams(use_tc_tiling_on_sc=False),
        out_type=jax.ShapeDtypeStruct((B, F * E), jnp.float32),
        scratch_types=[
            pltpu.VMEM((F, _CB), jnp.int32),
            pltpu.VMEM((F, _CB, E), jnp.float32),
            pltpu.SemaphoreType.DMA,
            pltpu.SemaphoreType.DMA,
        ],
    )
    def gather_k(table3_hbm, idxt_hbm, out_hbm, idx_v, big3, sem, wsem):
        wid = lax.axis_index("s") * _NC + lax.axis_index("c")
        b0 = wid * _BPW

        @pl.loop(0, _NCHUNK)
        def _chunk(c):
            cb = b0 + c * _CB
            pltpu.sync_copy(idxt_hbm.at[:, pl.ds(cb, _CB)], idx_v)
            for f in range(F):
                pltpu.async_copy(
                    table3_hbm.at[f].at[idx_v.at[f]], big3.at[f], sem
                )
            for f in range(F):
                pltpu.make_async_copy(
                    table3_hbm.at[f].at[idx_v.at[f]], big3.at[f], sem
                ).wait()
            for f in range(F):
                pltpu.async_copy(
                    big3.at[f],
                    out_hbm.at[pl.ds(cb, _CB), pl.ds(f * E, E)],
                    wsem,
                )
            for f in range(F):
                pltpu.make_async_copy(
                    big3.at[f],
                    out_hbm.at[pl.ds(cb, _CB), pl.ds(f * E, E)],
                    wsem,
                ).wait()

    return gather_k


_sc_gather = _build_gather()


_BB = 1024
_NB = B // _BB


def _gelu(x):
    return 0.5 * x * (1.0 + lax.erf(x * 0.7071067811865476))


def _mlp1_body(emb_ref, xn_ref, w1e_ref, w1n_ref, b1_ref, h1_ref, stats_ref,
               acc_ref):
    i = pl.program_id(0)

    @pl.when(i == 0)
    def _():
        acc_ref[...] = jnp.zeros_like(acc_ref)

    dn = (((1,), (1,)), ((), ()))
    h = (lax.dot_general(emb_ref[...], w1e_ref[...], dn,
                         preferred_element_type=jnp.float32)
         + lax.dot_general(xn_ref[...], w1n_ref[...], dn,
                           preferred_element_type=jnp.float32)
         + b1_ref[...])
    h1_ref[...] = h
    acc_ref[...] += jnp.concatenate(
        [jnp.sum(h, axis=0, keepdims=True),
         jnp.sum(h * h, axis=0, keepdims=True)], axis=0)

    @pl.when(i == _NB - 1)
    def _():
        stats_ref[...] = acc_ref[...]


def _mlp2_body(h1_ref, stats_ref, g1_ref, be1_ref, w2_ref, b2_ref, h2_ref,
               stats2_ref, acc_ref):
    i = pl.program_id(0)

    @pl.when(i == 0)
    def _():
        acc_ref[...] = jnp.zeros_like(acc_ref)

    mu = stats_ref[0:1, :] * (1.0 / B)
    var = stats_ref[1:2, :] * (1.0 / B) - mu * mu
    inv = lax.rsqrt(var + 1e-5)
    a = _gelu((h1_ref[...] - mu) * (inv * g1_ref[...]) + be1_ref[...])
    dn = (((1,), (1,)), ((), ()))
    h = (lax.dot_general(a, w2_ref[...], dn,
                         preferred_element_type=jnp.float32) + b2_ref[...])
    h2_ref[...] = h
    acc_ref[...] += jnp.concatenate(
        [jnp.sum(h, axis=0, keepdims=True),
         jnp.sum(h * h, axis=0, keepdims=True)], axis=0)

    @pl.when(i == _NB - 1)
    def _():
        stats2_ref[...] = acc_ref[...]


def _mlp3_body(h2_ref, stats2_ref, g2_ref, be2_ref, w3_ref, b3_ref, out_ref):
    mu = stats2_ref[0:1, :] * (1.0 / B)
    var = stats2_ref[1:2, :] * (1.0 / B) - mu * mu
    inv = lax.rsqrt(var + 1e-5)
    a = _gelu((h2_ref[...] - mu) * (inv * g2_ref[...]) + be2_ref[...])
    o = jnp.sum(a * w3_ref[...], axis=1, keepdims=True)
    out_ref[...] = o + b3_ref[...]


def _full(shape):
    return pl.BlockSpec(shape, lambda i: (0, 0))


_seq = pltpu.CompilerParams(dimension_semantics=("arbitrary",))


def kernel(x_num, tables, W1, b1, g1, be1, W2, b2, g2, be2, W3, b3, x_cat):
    idx_t = x_cat.astype(jnp.int32).T
    emb = _sc_gather(tables, idx_t)

    W1e = W1[:, :F * E]
    W1n = W1[:, F * E:]
    b1r = b1.reshape(1, H1)
    g1r = g1.reshape(1, H1)
    be1r = be1.reshape(1, H1)
    b2r = b2.reshape(1, H2)
    g2r = g2.reshape(1, H2)
    be2r = be2.reshape(1, H2)
    b3r = b3.reshape(1, 1)

    h1, stats1 = pl.pallas_call(
        _mlp1_body,
        grid=(_NB,),
        in_specs=[
            pl.BlockSpec((_BB, F * E), lambda i: (i, 0)),
            pl.BlockSpec((_BB, ND), lambda i: (i, 0)),
            _full((H1, F * E)),
            _full((H1, ND)),
            _full((1, H1)),
        ],
        out_specs=[
            pl.BlockSpec((_BB, H1), lambda i: (i, 0)),
            _full((2, H1)),
        ],
        out_shape=[
            jax.ShapeDtypeStruct((B, H1), jnp.float32),
            jax.ShapeDtypeStruct((2, H1), jnp.float32),
        ],
        scratch_shapes=[pltpu.VMEM((2, H1), jnp.float32)],
        compiler_params=_seq,
    )(emb, x_num, W1e, W1n, b1r)

    h2, stats2 = pl.pallas_call(
        _mlp2_body,
        grid=(_NB,),
        in_specs=[
            pl.BlockSpec((_BB, H1), lambda i: (i, 0)),
            _full((2, H1)),
            _full((1, H1)),
            _full((1, H1)),
            _full((H2, H1)),
            _full((1, H2)),
        ],
        out_specs=[
            pl.BlockSpec((_BB, H2), lambda i: (i, 0)),
            _full((2, H2)),
        ],
        out_shape=[
            jax.ShapeDtypeStruct((B, H2), jnp.float32),
            jax.ShapeDtypeStruct((2, H2), jnp.float32),
        ],
        scratch_shapes=[pltpu.VMEM((2, H2), jnp.float32)],
        compiler_params=_seq,
    )(h1, stats1, g1r, be1r, W2, b2r)

    out = pl.pallas_call(
        _mlp3_body,
        grid=(_NB,),
        in_specs=[
            pl.BlockSpec((_BB, H2), lambda i: (i, 0)),
            _full((2, H2)),
            _full((1, H2)),
            _full((1, H2)),
            _full((1, H2)),
            _full((1, 1)),
        ],
        out_specs=pl.BlockSpec((_BB, 1), lambda i: (i, 0)),
        out_shape=jax.ShapeDtypeStruct((B, 1), jnp.float32),
        compiler_params=_seq,
    )(h2, stats2, g2r, be2r, W3, b3r)

    return out.reshape(B)


# speedup vs baseline: 3.7353x; 3.7353x over previous
import functools

import jax
import jax.numpy as jnp
from jax import lax
from jax.experimental import pallas as pl
from jax.experimental.pallas import tpu as pltpu
from jax.experimental.pallas import tpu_sc as plsc

B = 16384
F = 26
V = 100000
E = 16
ND = 13
H1, H2 = 256, 128

_NC = 2
_NS = 16
_NW = _NC * _NS
_P = F * E
_PPW = _P // _NW
_BC = 4096
_NBC = B // _BC


def _build_gather():
    mesh = plsc.VectorSubcoreMesh(core_axis_name="c", subcore_axis_name="s")

    @functools.partial(
        pl.kernel,
        mesh=mesh,
        compiler_params=pltpu.CompilerParams(needs_layout_passes=False),
        out_type=jax.ShapeDtypeStruct((_P, B), jnp.float32),
        scratch_types=[
            pltpu.VMEM((V,), jnp.float32),
            pltpu.VMEM((_BC,), jnp.int32),
            pltpu.VMEM((_BC,), jnp.float32),
        ],
    )
    def gather_k(tt_hbm, idxt_hbm, out_hbm, plane_v, idx_v, ob):
        wid = lax.axis_index("s") * _NC + lax.axis_index("c")
        p0 = wid * _PPW

        @pl.loop(0, _PPW)
        def _plane(p):
            pe = p0 + p
            f = pe // E
            e = pe % E
            pltpu.sync_copy(tt_hbm.at[f, e], plane_v)

            @pl.loop(0, _NBC)
            def _bchunk(c):
                pltpu.sync_copy(idxt_hbm.at[f, pl.ds(c * _BC, _BC)], idx_v)

                @pl.loop(0, _BC // 16, unroll=4)
                def _g(i):
                    ii = idx_v[pl.ds(i * 16, 16)]
                    ob[pl.ds(i * 16, 16)] = plsc.load_gather(plane_v, [ii])

                pltpu.sync_copy(ob, out_hbm.at[pe, pl.ds(c * _BC, _BC)])

    return gather_k


_sc_gather = _build_gather()


_BB = 1024
_NB = B // _BB


def _gelu(x):
    return 0.5 * x * (1.0 + lax.erf(x * 0.7071067811865476))


def _mlp1_body(emb_ref, xn_ref, w1e_ref, w1n_ref, b1_ref, h1_ref, stats_ref,
               acc_ref):
    i = pl.program_id(0)

    @pl.when(i == 0)
    def _():
        acc_ref[...] = jnp.zeros_like(acc_ref)

    h = (lax.dot_general(w1e_ref[...], emb_ref[...], (((1,), (0,)), ((), ())),
                         preferred_element_type=jnp.float32)
         + lax.dot_general(w1n_ref[...], xn_ref[...], (((1,), (1,)), ((), ())),
                           preferred_element_type=jnp.float32)
         + b1_ref[...])
    h1_ref[...] = h
    acc_ref[...] += jnp.concatenate(
        [jnp.sum(h, axis=1, keepdims=True),
         jnp.sum(h * h, axis=1, keepdims=True)], axis=1)

    @pl.when(i == _NB - 1)
    def _():
        stats_ref[...] = acc_ref[...]


def _mlp2_body(h1_ref, stats_ref, g1_ref, be1_ref, w2_ref, b2_ref, h2_ref,
               stats2_ref, acc_ref):
    i = pl.program_id(0)

    @pl.when(i == 0)
    def _():
        acc_ref[...] = jnp.zeros_like(acc_ref)

    mu = stats_ref[:, 0:1] * (1.0 / B)
    var = stats_ref[:, 1:2] * (1.0 / B) - mu * mu
    inv = lax.rsqrt(var + 1e-5)
    a = _gelu((h1_ref[...] - mu) * (inv * g1_ref[...]) + be1_ref[...])
    h = (lax.dot_general(w2_ref[...], a, (((1,), (0,)), ((), ())),
                         preferred_element_type=jnp.float32) + b2_ref[...])
    h2_ref[...] = h
    acc_ref[...] += jnp.concatenate(
        [jnp.sum(h, axis=1, keepdims=True),
         jnp.sum(h * h, axis=1, keepdims=True)], axis=1)

    @pl.when(i == _NB - 1)
    def _():
        stats2_ref[...] = acc_ref[...]


def _mlp3_body(h2_ref, stats2_ref, g2_ref, be2_ref, w3_ref, b3_ref, out_ref):
    mu = stats2_ref[:, 0:1] * (1.0 / B)
    var = stats2_ref[:, 1:2] * (1.0 / B) - mu * mu
    inv = lax.rsqrt(var + 1e-5)
    a = _gelu((h2_ref[...] - mu) * (inv * g2_ref[...]) + be2_ref[...])
    o = jnp.sum(a * w3_ref[...], axis=0, keepdims=True)
    out_ref[...] = o + b3_ref[...]


def _full(shape):
    return pl.BlockSpec(shape, lambda i: (0, 0))


_seq = pltpu.CompilerParams(dimension_semantics=("arbitrary",))


def kernel(x_num, tables, W1, b1, g1, be1, W2, b2, g2, be2, W3, b3, x_cat):
    tt = jnp.swapaxes(tables, 1, 2)
    idx_t = x_cat.astype(jnp.int32).T
    emb_t = _sc_gather(tt, idx_t)

    W1e = W1[:, :F * E]
    W1n = W1[:, F * E:]
    b1c = b1.reshape(H1, 1)
    g1c = g1.reshape(H1, 1)
    be1c = be1.reshape(H1, 1)
    b2c = b2.reshape(H2, 1)
    g2c = g2.reshape(H2, 1)
    be2c = be2.reshape(H2, 1)
    w3c = W3.reshape(H2, 1)
    b3c = b3.reshape(1, 1)

    h1, stats1 = pl.pallas_call(
        _mlp1_body,
        grid=(_NB,),
        in_specs=[
            pl.BlockSpec((_P, _BB), lambda i: (0, i)),
            pl.BlockSpec((_BB, ND), lambda i: (i, 0)),
            _full((H1, _P)),
            _full((H1, ND)),
            _full((H1, 1)),
        ],
        out_specs=[
            pl.BlockSpec((H1, _BB), lambda i: (0, i)),
            _full((H1, 2)),
        ],
        out_shape=[
            jax.ShapeDtypeStruct((H1, B), jnp.float32),
            jax.ShapeDtypeStruct((H1, 2), jnp.float32),
        ],
        scratch_shapes=[pltpu.VMEM((H1, 2), jnp.float32)],
        compiler_params=_seq,
    )(emb_t, x_num, W1e, W1n, b1c)

    h2, stats2 = pl.pallas_call(
        _mlp2_body,
        grid=(_NB,),
        in_specs=[
            pl.BlockSpec((H1, _BB), lambda i: (0, i)),
            _full((H1, 2)),
            _full((H1, 1)),
            _full((H1, 1)),
            _full((H2, H1)),
            _full((H2, 1)),
        ],
        out_specs=[
            pl.BlockSpec((H2, _BB), lambda i: (0, i)),
            _full((H2, 2)),
        ],
        out_shape=[
            jax.ShapeDtypeStruct((H2, B), jnp.float32),
            jax.ShapeDtypeStruct((H2, 2), jnp.float32),
        ],
        scratch_shapes=[pltpu.VMEM((H2, 2), jnp.float32)],
        compiler_params=_seq,
    )(h1, stats1, g1c, be1c, W2, b2c)

    out = pl.pallas_call(
        _mlp3_body,
        grid=(_NB,),
        in_specs=[
            pl.BlockSpec((H2, _BB), lambda i: (0, i)),
            _full((H2, 2)),
            _full((H2, 1)),
            _full((H2, 1)),
            _full((H2, 1)),
            _full((1, 1)),
        ],
        out_specs=pl.BlockSpec((1, _BB), lambda i: (0, i)),
        out_shape=jax.ShapeDtypeStruct((1, B), jnp.float32),
        compiler_params=_seq,
    )(h2, stats2, g2c, be2c, w3c, b3c)

    return out.reshape(B)


# speedup vs baseline: 4.4798x; 1.1993x over previous
import functools

import jax
import jax.numpy as jnp
from jax import lax
from jax.experimental import pallas as pl
from jax.experimental.pallas import tpu as pltpu
from jax.experimental.pallas import tpu_sc as plsc

B = 16384
F = 26
V = 100000
E = 16
ND = 13
H1, H2 = 256, 128

_NC = 2
_NS = 16
_NW = _NC * _NS
_P = F * E
_PPW = _P // _NW
_BC = 4096
_NBC = B // _BC


def _build_gather():
    mesh = plsc.VectorSubcoreMesh(core_axis_name="c", subcore_axis_name="s")

    @functools.partial(
        pl.kernel,
        mesh=mesh,
        compiler_params=pltpu.CompilerParams(needs_layout_passes=False),
        out_type=jax.ShapeDtypeStruct((_P, B), jnp.float32),
        scratch_types=[
            pltpu.VMEM((V,), jnp.float32),
            pltpu.VMEM((_BC,), jnp.int32),
            pltpu.VMEM((_BC,), jnp.int32),
            pltpu.VMEM((_BC,), jnp.float32),
            pltpu.VMEM((_BC,), jnp.float32),
            pltpu.SemaphoreType.DMA,
            pltpu.SemaphoreType.DMA,
            pltpu.SemaphoreType.DMA,
            pltpu.SemaphoreType.DMA,
            pltpu.SemaphoreType.DMA,
        ],
    )
    def gather_k(tt_hbm, idxt_hbm, out_hbm, plane_v, idx0, idx1, ob0, ob1,
                 psem, isem0, isem1, osem0, osem1):
        wid = lax.axis_index("s") * _NC + lax.axis_index("c")
        p0 = wid * _PPW
        ivs = (idx0, idx1)
        obs = (ob0, ob1)
        isems = (isem0, isem1)
        osems = (osem0, osem1)

        @pl.loop(0, _PPW)
        def _plane(p):
            pe = p0 + p
            f = pe // E
            e = pe % E
            pltpu.async_copy(tt_hbm.at[f, e], plane_v, psem)
            pltpu.async_copy(idxt_hbm.at[f, pl.ds(0, _BC)], ivs[0], isems[0])

            @pl.when(p > 0)
            def _():
                for b in range(2):
                    pltpu.make_async_copy(
                        obs[b], out_hbm.at[pe, pl.ds(0, _BC)], osems[b]
                    ).wait()

            pltpu.make_async_copy(tt_hbm.at[f, e], plane_v, psem).wait()

            @pl.loop(0, _NBC, step=2)
            def _c2(c0):
                for b in range(2):
                    c = c0 + b
                    pltpu.make_async_copy(
                        idxt_hbm.at[f, pl.ds(0, _BC)], ivs[b], isems[b]
                    ).wait()

                    @pl.when(c + 1 < _NBC)
                    def _():
                        pltpu.async_copy(
                            idxt_hbm.at[f, pl.ds((c + 1) * _BC, _BC)],
                            ivs[1 - b], isems[1 - b])

                    @pl.when(c >= 2)
                    def _():
                        pltpu.make_async_copy(
                            obs[b], out_hbm.at[pe, pl.ds(0, _BC)], osems[b]
                        ).wait()

                    @pl.loop(0, _BC // 16, unroll=8)
                    def _g(i):
                        ii = ivs[b][pl.ds(i * 16, 16)]
                        obs[b][pl.ds(i * 16, 16)] = plsc.load_gather(
                            plane_v, [ii])

                    pltpu.async_copy(
                        obs[b], out_hbm.at[pe, pl.ds(c * _BC, _BC)], osems[b])

        for b in range(2):
            pltpu.make_async_copy(
                obs[b], out_hbm.at[0, pl.ds(0, _BC)], osems[b]
            ).wait()

    return gather_k


_sc_gather = _build_gather()


_BB = 1024
_NB = B // _BB


def _gelu(x):
    return 0.5 * x * (1.0 + lax.erf(x * 0.7071067811865476))


def _mlp1_body(emb_ref, xn_ref, w1e_ref, w1n_ref, b1_ref, h1_ref, stats_ref,
               acc_ref):
    i = pl.program_id(0)

    @pl.when(i == 0)
    def _():
        acc_ref[...] = jnp.zeros_like(acc_ref)

    h = (lax.dot_general(w1e_ref[...], emb_ref[...], (((1,), (0,)), ((), ())),
                         preferred_element_type=jnp.float32)
         + lax.dot_general(w1n_ref[...], xn_ref[...], (((1,), (1,)), ((), ())),
                           preferred_element_type=jnp.float32)
         + b1_ref[...])
    h1_ref[...] = h
    acc_ref[...] += jnp.concatenate(
        [jnp.sum(h, axis=1, keepdims=True),
         jnp.sum(h * h, axis=1, keepdims=True)], axis=1)

    @pl.when(i == _NB - 1)
    def _():
        stats_ref[...] = acc_ref[...]


def _mlp2_body(h1_ref, stats_ref, g1_ref, be1_ref, w2_ref, b2_ref, h2_ref,
               stats2_ref, acc_ref):
    i = pl.program_id(0)

    @pl.when(i == 0)
    def _():
        acc_ref[...] = jnp.zeros_like(acc_ref)

    mu = stats_ref[:, 0:1] * (1.0 / B)
    var = stats_ref[:, 1:2] * (1.0 / B) - mu * mu
    inv = lax.rsqrt(var + 1e-5)
    a = _gelu((h1_ref[...] - mu) * (inv * g1_ref[...]) + be1_ref[...])
    h = (lax.dot_general(w2_ref[...], a, (((1,), (0,)), ((), ())),
                         preferred_element_type=jnp.float32) + b2_ref[...])
    h2_ref[...] = h
    acc_ref[...] += jnp.concatenate(
        [jnp.sum(h, axis=1, keepdims=True),
         jnp.sum(h * h, axis=1, keepdims=True)], axis=1)

    @pl.when(i == _NB - 1)
    def _():
        stats2_ref[...] = acc_ref[...]


def _mlp3_body(h2_ref, stats2_ref, g2_ref, be2_ref, w3_ref, b3_ref, out_ref):
    mu = stats2_ref[:, 0:1] * (1.0 / B)
    var = stats2_ref[:, 1:2] * (1.0 / B) - mu * mu
    inv = lax.rsqrt(var + 1e-5)
    a = _gelu((h2_ref[...] - mu) * (inv * g2_ref[...]) + be2_ref[...])
    o = jnp.sum(a * w3_ref[...], axis=0, keepdims=True)
    out_ref[...] = o + b3_ref[...]


def _full(shape):
    return pl.BlockSpec(shape, lambda i: (0, 0))


_seq = pltpu.CompilerParams(dimension_semantics=("arbitrary",))


def kernel(x_num, tables, W1, b1, g1, be1, W2, b2, g2, be2, W3, b3, x_cat):
    tt = jnp.swapaxes(tables, 1, 2)
    idx_t = x_cat.astype(jnp.int32).T
    emb_t = _sc_gather(tt, idx_t)

    W1e = W1[:, :F * E]
    W1n = W1[:, F * E:]
    b1c = b1.reshape(H1, 1)
    g1c = g1.reshape(H1, 1)
    be1c = be1.reshape(H1, 1)
    b2c = b2.reshape(H2, 1)
    g2c = g2.reshape(H2, 1)
    be2c = be2.reshape(H2, 1)
    w3c = W3.reshape(H2, 1)
    b3c = b3.reshape(1, 1)

    h1, stats1 = pl.pallas_call(
        _mlp1_body,
        grid=(_NB,),
        in_specs=[
            pl.BlockSpec((_P, _BB), lambda i: (0, i)),
            pl.BlockSpec((_BB, ND), lambda i: (i, 0)),
            _full((H1, _P)),
            _full((H1, ND)),
            _full((H1, 1)),
        ],
        out_specs=[
            pl.BlockSpec((H1, _BB), lambda i: (0, i)),
            _full((H1, 2)),
        ],
        out_shape=[
            jax.ShapeDtypeStruct((H1, B), jnp.float32),
            jax.ShapeDtypeStruct((H1, 2), jnp.float32),
        ],
        scratch_shapes=[pltpu.VMEM((H1, 2), jnp.float32)],
        compiler_params=_seq,
    )(emb_t, x_num, W1e, W1n, b1c)

    h2, stats2 = pl.pallas_call(
        _mlp2_body,
        grid=(_NB,),
        in_specs=[
            pl.BlockSpec((H1, _BB), lambda i: (0, i)),
            _full((H1, 2)),
            _full((H1, 1)),
            _full((H1, 1)),
            _full((H2, H1)),
            _full((H2, 1)),
        ],
        out_specs=[
            pl.BlockSpec((H2, _BB), lambda i: (0, i)),
            _full((H2, 2)),
        ],
        out_shape=[
            jax.ShapeDtypeStruct((H2, B), jnp.float32),
            jax.ShapeDtypeStruct((H2, 2), jnp.float32),
        ],
        scratch_shapes=[pltpu.VMEM((H2, 2), jnp.float32)],
        compiler_params=_seq,
    )(h1, stats1, g1c, be1c, W2, b2c)

    out = pl.pallas_call(
        _mlp3_body,
        grid=(_NB,),
        in_specs=[
            pl.BlockSpec((H2, _BB), lambda i: (0, i)),
            _full((H2, 2)),
            _full((H2, 1)),
            _full((H2, 1)),
            _full((H2, 1)),
            _full((1, 1)),
        ],
        out_specs=pl.BlockSpec((1, _BB), lambda i: (0, i)),
        out_shape=jax.ShapeDtypeStruct((1, B), jnp.float32),
        compiler_params=_seq,
    )(h2, stats2, g2c, be2c, w3c, b3c)

    return out.reshape(B)


# speedup vs baseline: 6.3755x; 1.4232x over previous
import functools

import jax
import jax.numpy as jnp
from jax import lax
from jax.experimental import pallas as pl
from jax.experimental.pallas import tpu as pltpu
from jax.experimental.pallas import tpu_sc as plsc

B = 16384
F = 26
V = 100000
E = 16
ND = 13
H1, H2 = 256, 128

_NC = 2
_NS = 16
_NW = _NC * _NS
_P = F * E
_PPW = _P // _NW
_BC = 4096
_NBC = B // _BC


def _build_gather():
    mesh = plsc.VectorSubcoreMesh(core_axis_name="c", subcore_axis_name="s")

    @functools.partial(
        pl.kernel,
        mesh=mesh,
        compiler_params=pltpu.CompilerParams(needs_layout_passes=False),
        out_type=jax.ShapeDtypeStruct((_P, B), jnp.float32),
        scratch_types=[
            pltpu.VMEM((V,), jnp.float32),
            pltpu.VMEM((_BC,), jnp.int32),
            pltpu.VMEM((_BC,), jnp.int32),
            pltpu.VMEM((_BC,), jnp.float32),
            pltpu.VMEM((_BC,), jnp.float32),
            pltpu.SemaphoreType.DMA,
            pltpu.SemaphoreType.DMA,
            pltpu.SemaphoreType.DMA,
            pltpu.SemaphoreType.DMA,
            pltpu.SemaphoreType.DMA,
        ],
    )
    def gather_k(tt_hbm, idxt_hbm, out_hbm, plane_v, idx0, idx1, ob0, ob1,
                 psem, isem0, isem1, osem0, osem1):
        wid = lax.axis_index("s") * _NC + lax.axis_index("c")
        p0 = wid * _PPW
        ivs = (idx0, idx1)
        obs = (ob0, ob1)
        isems = (isem0, isem1)
        osems = (osem0, osem1)

        @pl.loop(0, _PPW)
        def _plane(p):
            pe = p0 + p
            f = pe // E
            e = pe % E
            pltpu.async_copy(tt_hbm.at[f, e], plane_v, psem)
            pltpu.async_copy(idxt_hbm.at[f, pl.ds(0, _BC)], ivs[0], isems[0])

            @pl.when(p > 0)
            def _():
                for b in range(2):
                    pltpu.make_async_copy(
                        obs[b], out_hbm.at[pe, pl.ds(0, _BC)], osems[b]
                    ).wait()

            pltpu.make_async_copy(tt_hbm.at[f, e], plane_v, psem).wait()

            @pl.loop(0, _NBC, step=2)
            def _c2(c0):
                for b in range(2):
                    c = c0 + b
                    pltpu.make_async_copy(
                        idxt_hbm.at[f, pl.ds(0, _BC)], ivs[b], isems[b]
                    ).wait()

                    @pl.when(c + 1 < _NBC)
                    def _():
                        pltpu.async_copy(
                            idxt_hbm.at[f, pl.ds((c + 1) * _BC, _BC)],
                            ivs[1 - b], isems[1 - b])

                    @pl.when(c >= 2)
                    def _():
                        pltpu.make_async_copy(
                            obs[b], out_hbm.at[pe, pl.ds(0, _BC)], osems[b]
                        ).wait()

                    @plsc.parallel_loop(0, _BC // 16, unroll=8)
                    def _g(i):
                        ii = ivs[b][pl.ds(i * 16, 16)]
                        obs[b][pl.ds(i * 16, 16)] = plsc.load_gather(
                            plane_v, [ii])

                    pltpu.async_copy(
                        obs[b], out_hbm.at[pe, pl.ds(c * _BC, _BC)], osems[b])

        for b in range(2):
            pltpu.make_async_copy(
                obs[b], out_hbm.at[0, pl.ds(0, _BC)], osems[b]
            ).wait()

    return gather_k


_sc_gather = _build_gather()


_BB = 1024
_NB = B // _BB


def _gelu(x):
    return 0.5 * x * (1.0 + lax.erf(x * 0.7071067811865476))


def _mlp1_body(emb_ref, xn_ref, w1e_ref, w1n_ref, b1_ref, h1_ref, stats_ref,
               acc_ref):
    i = pl.program_id(0)

    @pl.when(i == 0)
    def _():
        acc_ref[...] = jnp.zeros_like(acc_ref)

    h = (lax.dot_general(w1e_ref[...], emb_ref[...], (((1,), (0,)), ((), ())),
                         preferred_element_type=jnp.float32)
         + lax.dot_general(w1n_ref[...], xn_ref[...], (((1,), (1,)), ((), ())),
                           preferred_element_type=jnp.float32)
         + b1_ref[...])
    h1_ref[...] = h
    acc_ref[...] += jnp.concatenate(
        [jnp.sum(h, axis=1, keepdims=True),
         jnp.sum(h * h, axis=1, keepdims=True)], axis=1)

    @pl.when(i == _NB - 1)
    def _():
        stats_ref[...] = acc_ref[...]


def _mlp2_body(h1_ref, stats_ref, g1_ref, be1_ref, w2_ref, b2_ref, h2_ref,
               stats2_ref, acc_ref):
    i = pl.program_id(0)

    @pl.when(i == 0)
    def _():
        acc_ref[...] = jnp.zeros_like(acc_ref)

    mu = stats_ref[:, 0:1] * (1.0 / B)
    var = stats_ref[:, 1:2] * (1.0 / B) - mu * mu
    inv = lax.rsqrt(var + 1e-5)
    a = _gelu((h1_ref[...] - mu) * (inv * g1_ref[...]) + be1_ref[...])
    h = (lax.dot_general(w2_ref[...], a, (((1,), (0,)), ((), ())),
                         preferred_element_type=jnp.float32) + b2_ref[...])
    h2_ref[...] = h
    acc_ref[...] += jnp.concatenate(
        [jnp.sum(h, axis=1, keepdims=True),
         jnp.sum(h * h, axis=1, keepdims=True)], axis=1)

    @pl.when(i == _NB - 1)
    def _():
        stats2_ref[...] = acc_ref[...]


def _mlp3_body(h2_ref, stats2_ref, g2_ref, be2_ref, w3_ref, b3_ref, out_ref):
    mu = stats2_ref[:, 0:1] * (1.0 / B)
    var = stats2_ref[:, 1:2] * (1.0 / B) - mu * mu
    inv = lax.rsqrt(var + 1e-5)
    a = _gelu((h2_ref[...] - mu) * (inv * g2_ref[...]) + be2_ref[...])
    o = jnp.sum(a * w3_ref[...], axis=0, keepdims=True)
    out_ref[...] = o + b3_ref[...]


def _full(shape):
    return pl.BlockSpec(shape, lambda i: (0, 0))


_seq = pltpu.CompilerParams(dimension_semantics=("arbitrary",))


def kernel(x_num, tables, W1, b1, g1, be1, W2, b2, g2, be2, W3, b3, x_cat):
    tt = jnp.swapaxes(tables, 1, 2)
    idx_t = x_cat.astype(jnp.int32).T
    emb_t = _sc_gather(tt, idx_t)

    W1e = W1[:, :F * E]
    W1n = W1[:, F * E:]
    b1c = b1.reshape(H1, 1)
    g1c = g1.reshape(H1, 1)
    be1c = be1.reshape(H1, 1)
    b2c = b2.reshape(H2, 1)
    g2c = g2.reshape(H2, 1)
    be2c = be2.reshape(H2, 1)
    w3c = W3.reshape(H2, 1)
    b3c = b3.reshape(1, 1)

    h1, stats1 = pl.pallas_call(
        _mlp1_body,
        grid=(_NB,),
        in_specs=[
            pl.BlockSpec((_P, _BB), lambda i: (0, i)),
            pl.BlockSpec((_BB, ND), lambda i: (i, 0)),
            _full((H1, _P)),
            _full((H1, ND)),
            _full((H1, 1)),
        ],
        out_specs=[
            pl.BlockSpec((H1, _BB), lambda i: (0, i)),
            _full((H1, 2)),
        ],
        out_shape=[
            jax.ShapeDtypeStruct((H1, B), jnp.float32),
            jax.ShapeDtypeStruct((H1, 2), jnp.float32),
        ],
        scratch_shapes=[pltpu.VMEM((H1, 2), jnp.float32)],
        compiler_params=_seq,
    )(emb_t, x_num, W1e, W1n, b1c)

    h2, stats2 = pl.pallas_call(
        _mlp2_body,
        grid=(_NB,),
        in_specs=[
            pl.BlockSpec((H1, _BB), lambda i: (0, i)),
            _full((H1, 2)),
            _full((H1, 1)),
            _full((H1, 1)),
            _full((H2, H1)),
            _full((H2, 1)),
        ],
        out_specs=[
            pl.BlockSpec((H2, _BB), lambda i: (0, i)),
            _full((H2, 2)),
        ],
        out_shape=[
            jax.ShapeDtypeStruct((H2, B), jnp.float32),
            jax.ShapeDtypeStruct((H2, 2), jnp.float32),
        ],
        scratch_shapes=[pltpu.VMEM((H2, 2), jnp.float32)],
        compiler_params=_seq,
    )(h1, stats1, g1c, be1c, W2, b2c)

    out = pl.pallas_call(
        _mlp3_body,
        grid=(_NB,),
        in_specs=[
            pl.BlockSpec((H2, _BB), lambda i: (0, i)),
            _full((H2, 2)),
            _full((H2, 1)),
            _full((H2, 1)),
            _full((H2, 1)),
            _full((1, 1)),
        ],
        out_specs=pl.BlockSpec((1, _BB), lambda i: (0, i)),
        out_shape=jax.ShapeDtypeStruct((1, B), jnp.float32),
        compiler_params=_seq,
    )(h2, stats2, g2c, be2c, w3c, b3c)

    return out.reshape(B)


# speedup vs baseline: 7.1600x; 1.1230x over previous
import functools

import jax
import jax.numpy as jnp
from jax import lax
from jax.experimental import pallas as pl
from jax.experimental.pallas import tpu as pltpu
from jax.experimental.pallas import tpu_sc as plsc

B = 16384
F = 26
V = 100000
E = 16
ND = 13
H1, H2 = 256, 128

_NC = 2
_NS = 16
_NW = _NC * _NS
_P = F * E
_PPW = _P // _NW
_BC = 4096
_NBC = B // _BC


def _build_gather():
    mesh = plsc.VectorSubcoreMesh(core_axis_name="c", subcore_axis_name="s")

    @functools.partial(
        pl.kernel,
        mesh=mesh,
        compiler_params=pltpu.CompilerParams(needs_layout_passes=False),
        out_type=jax.ShapeDtypeStruct((_P, B), jnp.float32),
        scratch_types=[
            pltpu.VMEM((V,), jnp.float32),
            pltpu.VMEM((B,), jnp.int32),
            pltpu.VMEM((_BC,), jnp.float32),
            pltpu.VMEM((_BC,), jnp.float32),
            pltpu.SemaphoreType.DMA,
            pltpu.SemaphoreType.DMA,
            pltpu.SemaphoreType.DMA,
            pltpu.SemaphoreType.DMA,
        ],
    )
    def gather_k(tt_hbm, idxt_hbm, out_hbm, plane_v, idxf, ob0, ob1,
                 psem, isem, osem0, osem1):
        wid = lax.axis_index("s") * _NC + lax.axis_index("c")
        p0 = wid * _PPW
        obs = (ob0, ob1)
        osems = (osem0, osem1)

        @pl.loop(0, _PPW)
        def _plane(p):
            pe = p0 + p
            f = pe // E
            e = pe % E
            pltpu.async_copy(tt_hbm.at[f, e], plane_v, psem)
            new_f = jnp.logical_or(p == 0, e == 0)

            @pl.when(new_f)
            def _():
                pltpu.async_copy(idxt_hbm.at[f], idxf, isem)

            @pl.when(p > 0)
            def _():
                for b in range(2):
                    pltpu.make_async_copy(
                        obs[b], out_hbm.at[pe, pl.ds(0, _BC)], osems[b]
                    ).wait()

            @pl.when(new_f)
            def _():
                pltpu.make_async_copy(idxt_hbm.at[f], idxf, isem).wait()

            pltpu.make_async_copy(tt_hbm.at[f, e], plane_v, psem).wait()

            @pl.loop(0, _NBC, step=2)
            def _c2(c0):
                for b in range(2):
                    c = c0 + b

                    @pl.when(c >= 2)
                    def _():
                        pltpu.make_async_copy(
                            obs[b], out_hbm.at[pe, pl.ds(0, _BC)], osems[b]
                        ).wait()

                    base = c * _BC

                    @plsc.parallel_loop(0, _BC // 16, unroll=8)
                    def _g(i):
                        ii = idxf[pl.ds(base + i * 16, 16)]
                        obs[b][pl.ds(i * 16, 16)] = plsc.load_gather(
                            plane_v, [ii])

                    pltpu.async_copy(
                        obs[b], out_hbm.at[pe, pl.ds(c * _BC, _BC)], osems[b])

        for b in range(2):
            pltpu.make_async_copy(
                obs[b], out_hbm.at[0, pl.ds(0, _BC)], osems[b]
            ).wait()

    return gather_k


_sc_gather = _build_gather()


_BB = 1024
_NB = B // _BB


def _gelu(x):
    return 0.5 * x * (1.0 + lax.erf(x * 0.7071067811865476))


def _mlp1_body(emb_ref, xn_ref, w1e_ref, w1n_ref, b1_ref, h1_ref, stats_ref,
               acc_ref):
    i = pl.program_id(0)

    @pl.when(i == 0)
    def _():
        acc_ref[...] = jnp.zeros_like(acc_ref)

    h = (lax.dot_general(w1e_ref[...], emb_ref[...], (((1,), (0,)), ((), ())),
                         preferred_element_type=jnp.float32)
         + lax.dot_general(w1n_ref[...], xn_ref[...], (((1,), (1,)), ((), ())),
                           preferred_element_type=jnp.float32)
         + b1_ref[...])
    h1_ref[...] = h
    acc_ref[...] += jnp.concatenate(
        [jnp.sum(h, axis=1, keepdims=True),
         jnp.sum(h * h, axis=1, keepdims=True)], axis=1)

    @pl.when(i == _NB - 1)
    def _():
        stats_ref[...] = acc_ref[...]


def _mlp2_body(h1_ref, stats_ref, g1_ref, be1_ref, w2_ref, b2_ref, h2_ref,
               stats2_ref, acc_ref):
    i = pl.program_id(0)

    @pl.when(i == 0)
    def _():
        acc_ref[...] = jnp.zeros_like(acc_ref)

    mu = stats_ref[:, 0:1] * (1.0 / B)
    var = stats_ref[:, 1:2] * (1.0 / B) - mu * mu
    inv = lax.rsqrt(var + 1e-5)
    a = _gelu((h1_ref[...] - mu) * (inv * g1_ref[...]) + be1_ref[...])
    h = (lax.dot_general(w2_ref[...], a, (((1,), (0,)), ((), ())),
                         preferred_element_type=jnp.float32) + b2_ref[...])
    h2_ref[...] = h
    acc_ref[...] += jnp.concatenate(
        [jnp.sum(h, axis=1, keepdims=True),
         jnp.sum(h * h, axis=1, keepdims=True)], axis=1)

    @pl.when(i == _NB - 1)
    def _():
        stats2_ref[...] = acc_ref[...]


def _mlp3_body(h2_ref, stats2_ref, g2_ref, be2_ref, w3_ref, b3_ref, out_ref):
    mu = stats2_ref[:, 0:1] * (1.0 / B)
    var = stats2_ref[:, 1:2] * (1.0 / B) - mu * mu
    inv = lax.rsqrt(var + 1e-5)
    a = _gelu((h2_ref[...] - mu) * (inv * g2_ref[...]) + be2_ref[...])
    o = jnp.sum(a * w3_ref[...], axis=0, keepdims=True)
    out_ref[...] = o + b3_ref[...]


def _full(shape):
    return pl.BlockSpec(shape, lambda i: (0, 0))


_seq = pltpu.CompilerParams(dimension_semantics=("arbitrary",))


def kernel(x_num, tables, W1, b1, g1, be1, W2, b2, g2, be2, W3, b3, x_cat):
    tt = jnp.swapaxes(tables, 1, 2)
    idx_t = x_cat.astype(jnp.int32).T
    emb_t = _sc_gather(tt, idx_t)

    W1e = W1[:, :F * E]
    W1n = W1[:, F * E:]
    b1c = b1.reshape(H1, 1)
    g1c = g1.reshape(H1, 1)
    be1c = be1.reshape(H1, 1)
    b2c = b2.reshape(H2, 1)
    g2c = g2.reshape(H2, 1)
    be2c = be2.reshape(H2, 1)
    w3c = W3.reshape(H2, 1)
    b3c = b3.reshape(1, 1)

    h1, stats1 = pl.pallas_call(
        _mlp1_body,
        grid=(_NB,),
        in_specs=[
            pl.BlockSpec((_P, _BB), lambda i: (0, i)),
            pl.BlockSpec((_BB, ND), lambda i: (i, 0)),
            _full((H1, _P)),
            _full((H1, ND)),
            _full((H1, 1)),
        ],
        out_specs=[
            pl.BlockSpec((H1, _BB), lambda i: (0, i)),
            _full((H1, 2)),
        ],
        out_shape=[
            jax.ShapeDtypeStruct((H1, B), jnp.float32),
            jax.ShapeDtypeStruct((H1, 2), jnp.float32),
        ],
        scratch_shapes=[pltpu.VMEM((H1, 2), jnp.float32)],
        compiler_params=_seq,
    )(emb_t, x_num, W1e, W1n, b1c)

    h2, stats2 = pl.pallas_call(
        _mlp2_body,
        grid=(_NB,),
        in_specs=[
            pl.BlockSpec((H1, _BB), lambda i: (0, i)),
            _full((H1, 2)),
            _full((H1, 1)),
            _full((H1, 1)),
            _full((H2, H1)),
            _full((H2, 1)),
        ],
        out_specs=[
            pl.BlockSpec((H2, _BB), lambda i: (0, i)),
            _full((H2, 2)),
        ],
        out_shape=[
            jax.ShapeDtypeStruct((H2, B), jnp.float32),
            jax.ShapeDtypeStruct((H2, 2), jnp.float32),
        ],
        scratch_shapes=[pltpu.VMEM((H2, 2), jnp.float32)],
        compiler_params=_seq,
    )(h1, stats1, g1c, be1c, W2, b2c)

    out = pl.pallas_call(
        _mlp3_body,
        grid=(_NB,),
        in_specs=[
            pl.BlockSpec((H2, _BB), lambda i: (0, i)),
            _full((H2, 2)),
            _full((H2, 1)),
            _full((H2, 1)),
            _full((H2, 1)),
            _full((1, 1)),
        ],
        out_specs=pl.BlockSpec((1, _BB), lambda i: (0, i)),
        out_shape=jax.ShapeDtypeStruct((1, B), jnp.float32),
        compiler_params=_seq,
    )(h2, stats2, g2c, be2c, w3c, b3c)

    return out.reshape(B)


# speedup vs baseline: 7.5112x; 1.0491x over previous
import functools

import jax
import jax.numpy as jnp
from jax import lax
from jax.experimental import pallas as pl
from jax.experimental.pallas import tpu as pltpu
from jax.experimental.pallas import tpu_sc as plsc

B = 16384
F = 26
V = 100000
E = 16
ND = 13
H1, H2 = 256, 128

_NC = 2
_NS = 16
_NW = _NC * _NS
_P = F * E
_PPW = _P // _NW
_BC = 4096
_NBC = B // _BC


def _build_gather():
    mesh = plsc.VectorSubcoreMesh(core_axis_name="c", subcore_axis_name="s")

    @functools.partial(
        pl.kernel,
        mesh=mesh,
        compiler_params=pltpu.CompilerParams(needs_layout_passes=False),
        out_type=jax.ShapeDtypeStruct((_P, B), jnp.float32),
        scratch_types=[
            pltpu.VMEM((V,), jnp.float32),
            pltpu.VMEM((B,), jnp.int32),
            pltpu.VMEM((_BC,), jnp.float32),
            pltpu.VMEM((_BC,), jnp.float32),
            pltpu.SemaphoreType.DMA,
            pltpu.SemaphoreType.DMA,
            pltpu.SemaphoreType.DMA,
            pltpu.SemaphoreType.DMA,
        ],
    )
    def gather_k(tt_hbm, idxt_hbm, out_hbm, plane_v, idxf, ob0, ob1,
                 psem, isem, osem0, osem1):
        wid = lax.axis_index("s") * _NC + lax.axis_index("c")
        p0 = wid * _PPW
        obs = (ob0, ob1)
        osems = (osem0, osem1)

        @pl.loop(0, _PPW)
        def _plane(p):
            pe = p0 + p
            f = pe // E
            e = pe % E
            pltpu.async_copy(tt_hbm.at[f, e], plane_v, psem)
            new_f = jnp.logical_or(p == 0, e == 0)

            @pl.when(new_f)
            def _():
                pltpu.async_copy(idxt_hbm.at[f], idxf, isem)

            @pl.when(p > 0)
            def _():
                for b in range(2):
                    pltpu.make_async_copy(
                        obs[b], out_hbm.at[pe, pl.ds(0, _BC)], osems[b]
                    ).wait()

            @pl.when(new_f)
            def _():
                pltpu.make_async_copy(idxt_hbm.at[f], idxf, isem).wait()

            pltpu.make_async_copy(tt_hbm.at[f, e], plane_v, psem).wait()

            @pl.loop(0, _NBC, step=2)
            def _c2(c0):
                for b in range(2):
                    c = c0 + b

                    @pl.when(c >= 2)
                    def _():
                        pltpu.make_async_copy(
                            obs[b], out_hbm.at[pe, pl.ds(0, _BC)], osems[b]
                        ).wait()

                    base = c * _BC

                    @plsc.parallel_loop(0, _BC // 16, unroll=8)
                    def _g(i):
                        ii = idxf[pl.ds(base + i * 16, 16)]
                        obs[b][pl.ds(i * 16, 16)] = plsc.load_gather(
                            plane_v, [ii])

                    pltpu.async_copy(
                        obs[b], out_hbm.at[pe, pl.ds(c * _BC, _BC)], osems[b])

        for b in range(2):
            pltpu.make_async_copy(
                obs[b], out_hbm.at[0, pl.ds(0, _BC)], osems[b]
            ).wait()

    return gather_k


_sc_gather = _build_gather()


_BB = 1024
_NB = B // _BB


def _gelu(x):
    return 0.5 * x * (1.0 + lax.erf(x * 0.7071067811865476))


def _mlp1_body(emb_ref, xn_ref, w1e_ref, w1n_ref, b1_ref, h1_ref, stats_ref,
               acc_ref):
    i = pl.program_id(0)

    @pl.when(i == 0)
    def _():
        acc_ref[...] = jnp.zeros_like(acc_ref)

    h = (lax.dot_general(w1e_ref[...], emb_ref[...], (((1,), (0,)), ((), ())),
                         preferred_element_type=jnp.float32)
         + lax.dot_general(w1n_ref[...], xn_ref[...], (((1,), (1,)), ((), ())),
                           preferred_element_type=jnp.float32)
         + b1_ref[...])
    h1_ref[...] = h.astype(jnp.bfloat16)
    acc_ref[...] += jnp.concatenate(
        [jnp.sum(h, axis=1, keepdims=True),
         jnp.sum(h * h, axis=1, keepdims=True)], axis=1)

    @pl.when(i == _NB - 1)
    def _():
        stats_ref[...] = acc_ref[...]


def _mlp2_body(h1_ref, stats_ref, g1_ref, be1_ref, w2_ref, b2_ref, h2_ref,
               stats2_ref, acc_ref):
    i = pl.program_id(0)

    @pl.when(i == 0)
    def _():
        acc_ref[...] = jnp.zeros_like(acc_ref)

    mu = stats_ref[:, 0:1] * (1.0 / B)
    var = stats_ref[:, 1:2] * (1.0 / B) - mu * mu
    inv = lax.rsqrt(var + 1e-5)
    h1 = h1_ref[...].astype(jnp.float32)
    a = _gelu((h1 - mu) * (inv * g1_ref[...]) + be1_ref[...])
    h = (lax.dot_general(w2_ref[...], a, (((1,), (0,)), ((), ())),
                         preferred_element_type=jnp.float32) + b2_ref[...])
    h2_ref[...] = h.astype(jnp.bfloat16)
    acc_ref[...] += jnp.concatenate(
        [jnp.sum(h, axis=1, keepdims=True),
         jnp.sum(h * h, axis=1, keepdims=True)], axis=1)

    @pl.when(i == _NB - 1)
    def _():
        stats2_ref[...] = acc_ref[...]


def _mlp3_body(h2_ref, stats2_ref, g2_ref, be2_ref, w3_ref, b3_ref, out_ref):
    mu = stats2_ref[:, 0:1] * (1.0 / B)
    var = stats2_ref[:, 1:2] * (1.0 / B) - mu * mu
    inv = lax.rsqrt(var + 1e-5)
    h2 = h2_ref[...].astype(jnp.float32)
    a = _gelu((h2 - mu) * (inv * g2_ref[...]) + be2_ref[...])
    o = jnp.sum(a * w3_ref[...], axis=0, keepdims=True)
    out_ref[...] = o + b3_ref[...]


def _full(shape):
    return pl.BlockSpec(shape, lambda i: (0, 0))


_seq = pltpu.CompilerParams(dimension_semantics=("arbitrary",))


def kernel(x_num, tables, W1, b1, g1, be1, W2, b2, g2, be2, W3, b3, x_cat):
    tt = jnp.swapaxes(tables, 1, 2)
    idx_t = x_cat.astype(jnp.int32).T
    emb_t = _sc_gather(tt, idx_t)

    W1e = W1[:, :F * E]
    W1n = W1[:, F * E:]
    b1c = b1.reshape(H1, 1)
    g1c = g1.reshape(H1, 1)
    be1c = be1.reshape(H1, 1)
    b2c = b2.reshape(H2, 1)
    g2c = g2.reshape(H2, 1)
    be2c = be2.reshape(H2, 1)
    w3c = W3.reshape(H2, 1)
    b3c = b3.reshape(1, 1)

    h1, stats1 = pl.pallas_call(
        _mlp1_body,
        grid=(_NB,),
        in_specs=[
            pl.BlockSpec((_P, _BB), lambda i: (0, i)),
            pl.BlockSpec((_BB, ND), lambda i: (i, 0)),
            _full((H1, _P)),
            _full((H1, ND)),
            _full((H1, 1)),
        ],
        out_specs=[
            pl.BlockSpec((H1, _BB), lambda i: (0, i)),
            _full((H1, 2)),
        ],
        out_shape=[
            jax.ShapeDtypeStruct((H1, B), jnp.bfloat16),
            jax.ShapeDtypeStruct((H1, 2), jnp.float32),
        ],
        scratch_shapes=[pltpu.VMEM((H1, 2), jnp.float32)],
        compiler_params=_seq,
    )(emb_t, x_num, W1e, W1n, b1c)

    h2, stats2 = pl.pallas_call(
        _mlp2_body,
        grid=(_NB,),
        in_specs=[
            pl.BlockSpec((H1, _BB), lambda i: (0, i)),
            _full((H1, 2)),
            _full((H1, 1)),
            _full((H1, 1)),
            _full((H2, H1)),
            _full((H2, 1)),
        ],
        out_specs=[
            pl.BlockSpec((H2, _BB), lambda i: (0, i)),
            _full((H2, 2)),
        ],
        out_shape=[
            jax.ShapeDtypeStruct((H2, B), jnp.bfloat16),
            jax.ShapeDtypeStruct((H2, 2), jnp.float32),
        ],
        scratch_shapes=[pltpu.VMEM((H2, 2), jnp.float32)],
        compiler_params=_seq,
    )(h1, stats1, g1c, be1c, W2, b2c)

    out = pl.pallas_call(
        _mlp3_body,
        grid=(_NB // 2,),
        in_specs=[
            pl.BlockSpec((H2, 2 * _BB), lambda i: (0, i)),
            _full((H2, 2)),
            _full((H2, 1)),
            _full((H2, 1)),
            _full((H2, 1)),
            _full((1, 1)),
        ],
        out_specs=pl.BlockSpec((1, 2 * _BB), lambda i: (0, i)),
        out_shape=jax.ShapeDtypeStruct((1, B), jnp.float32),
        compiler_params=_seq,
    )(h2, stats2, g2c, be2c, w3c, b3c)

    return out.reshape(B)


# speedup vs baseline: 7.7338x; 1.0296x over previous
import functools

import jax
import jax.numpy as jnp
from jax import lax
from jax.experimental import pallas as pl
from jax.experimental.pallas import tpu as pltpu
from jax.experimental.pallas import tpu_sc as plsc

B = 16384
F = 26
V = 100000
E = 16
ND = 13
H1, H2 = 256, 128

_NC = 2
_NS = 16
_NW = _NC * _NS
_P = F * E
_PPW = _P // _NW
_BC = 4096
_NBC = B // _BC


def _build_gather():
    mesh = plsc.VectorSubcoreMesh(core_axis_name="c", subcore_axis_name="s")

    @functools.partial(
        pl.kernel,
        mesh=mesh,
        compiler_params=pltpu.CompilerParams(needs_layout_passes=False),
        out_type=jax.ShapeDtypeStruct((_P, B), jnp.float32),
        scratch_types=[
            pltpu.VMEM((V,), jnp.float32),
            pltpu.VMEM((B,), jnp.int32),
            pltpu.VMEM((_BC,), jnp.float32),
            pltpu.VMEM((_BC,), jnp.float32),
            pltpu.SemaphoreType.DMA,
            pltpu.SemaphoreType.DMA,
            pltpu.SemaphoreType.DMA,
            pltpu.SemaphoreType.DMA,
        ],
    )
    def gather_k(tt_hbm, idxt_hbm, out_hbm, plane_v, idxf, ob0, ob1,
                 psem, isem, osem0, osem1):
        wid = lax.axis_index("s") * _NC + lax.axis_index("c")
        p0 = wid * _PPW
        obs = (ob0, ob1)
        osems = (osem0, osem1)

        @pl.loop(0, _PPW)
        def _plane(p):
            pe = p0 + p
            f = pe // E
            e = pe % E
            pltpu.async_copy(tt_hbm.at[f, e], plane_v, psem)
            new_f = jnp.logical_or(p == 0, e == 0)

            @pl.when(new_f)
            def _():
                pltpu.async_copy(idxt_hbm.at[f], idxf, isem)

            @pl.when(p > 0)
            def _():
                for b in range(2):
                    pltpu.make_async_copy(
                        obs[b], out_hbm.at[pe, pl.ds(0, _BC)], osems[b]
                    ).wait()

            @pl.when(new_f)
            def _():
                pltpu.make_async_copy(idxt_hbm.at[f], idxf, isem).wait()

            pltpu.make_async_copy(tt_hbm.at[f, e], plane_v, psem).wait()

            @pl.loop(0, _NBC, step=2)
            def _c2(c0):
                for b in range(2):
                    c = c0 + b

                    @pl.when(c >= 2)
                    def _():
                        pltpu.make_async_copy(
                            obs[b], out_hbm.at[pe, pl.ds(0, _BC)], osems[b]
                        ).wait()

                    base = c * _BC

                    @plsc.parallel_loop(0, _BC // 16, unroll=8)
                    def _g(i):
                        ii = idxf[pl.ds(base + i * 16, 16)]
                        obs[b][pl.ds(i * 16, 16)] = plsc.load_gather(
                            plane_v, [ii])

                    pltpu.async_copy(
                        obs[b], out_hbm.at[pe, pl.ds(c * _BC, _BC)], osems[b])

        for b in range(2):
            pltpu.make_async_copy(
                obs[b], out_hbm.at[0, pl.ds(0, _BC)], osems[b]
            ).wait()

    return gather_k


_sc_gather = _build_gather()


_BB = 1024
_NB = B // _BB


def _gelu(x):
    return 0.5 * x * (1.0 + lax.erf(x * 0.7071067811865476))


def _mlp1_body(emb_ref, xn_ref, w1e_ref, w1n_ref, b1_ref, h1_ref, stats_ref,
               acc_ref):
    i = pl.program_id(0)

    @pl.when(i == 0)
    def _():
        acc_ref[...] = jnp.zeros_like(acc_ref)

    h = (lax.dot_general(w1e_ref[...], emb_ref[...], (((1,), (0,)), ((), ())),
                         preferred_element_type=jnp.float32)
         + lax.dot_general(w1n_ref[...], xn_ref[...], (((1,), (1,)), ((), ())),
                           preferred_element_type=jnp.float32)
         + b1_ref[...])
    h1_ref[...] = h.astype(jnp.bfloat16)
    acc_ref[...] += jnp.concatenate(
        [jnp.sum(h, axis=1, keepdims=True),
         jnp.sum(h * h, axis=1, keepdims=True)], axis=1)

    @pl.when(i == pl.num_programs(0) - 1)
    def _():
        stats_ref[...] = acc_ref[...]


def _mlp2_body(h1_ref, stats_ref, g1_ref, be1_ref, w2_ref, b2_ref, h2_ref,
               stats2_ref, acc_ref):
    i = pl.program_id(0)

    @pl.when(i == 0)
    def _():
        acc_ref[...] = jnp.zeros_like(acc_ref)

    mu = stats_ref[:, 0:1] * (1.0 / B)
    var = stats_ref[:, 1:2] * (1.0 / B) - mu * mu
    inv = lax.rsqrt(var + 1e-5)
    h1 = h1_ref[...].astype(jnp.float32)
    a = _gelu((h1 - mu) * (inv * g1_ref[...]) + be1_ref[...])
    h = (lax.dot_general(w2_ref[...], a, (((1,), (0,)), ((), ())),
                         preferred_element_type=jnp.float32) + b2_ref[...])
    h2_ref[...] = h.astype(jnp.bfloat16)
    acc_ref[...] += jnp.concatenate(
        [jnp.sum(h, axis=1, keepdims=True),
         jnp.sum(h * h, axis=1, keepdims=True)], axis=1)

    @pl.when(i == pl.num_programs(0) - 1)
    def _():
        stats2_ref[...] = acc_ref[...]


def _mlp3_body(h2_ref, stats2_ref, g2_ref, be2_ref, w3_ref, b3_ref, out_ref):
    mu = stats2_ref[:, 0:1] * (1.0 / B)
    var = stats2_ref[:, 1:2] * (1.0 / B) - mu * mu
    inv = lax.rsqrt(var + 1e-5)
    h2 = h2_ref[...].astype(jnp.float32)
    a = _gelu((h2 - mu) * (inv * g2_ref[...]) + be2_ref[...])
    o = jnp.sum(a * w3_ref[...], axis=0, keepdims=True)
    out_ref[...] = o + b3_ref[...]


def _full(shape):
    return pl.BlockSpec(shape, lambda i: (0, 0))


_seq = pltpu.CompilerParams(dimension_semantics=("arbitrary",))


def kernel(x_num, tables, W1, b1, g1, be1, W2, b2, g2, be2, W3, b3, x_cat):
    tt = jnp.swapaxes(tables, 1, 2)
    idx_t = x_cat.astype(jnp.int32).T
    emb_t = _sc_gather(tt, idx_t)

    W1e = W1[:, :F * E]
    W1n = W1[:, F * E:]
    b1c = b1.reshape(H1, 1)
    g1c = g1.reshape(H1, 1)
    be1c = be1.reshape(H1, 1)
    b2c = b2.reshape(H2, 1)
    g2c = g2.reshape(H2, 1)
    be2c = be2.reshape(H2, 1)
    w3c = W3.reshape(H2, 1)
    b3c = b3.reshape(1, 1)

    h1, stats1 = pl.pallas_call(
        _mlp1_body,
        grid=(_NB,),
        in_specs=[
            pl.BlockSpec((_P, _BB), lambda i: (0, i)),
            pl.BlockSpec((_BB, ND), lambda i: (i, 0)),
            _full((H1, _P)),
            _full((H1, ND)),
            _full((H1, 1)),
        ],
        out_specs=[
            pl.BlockSpec((H1, _BB), lambda i: (0, i)),
            _full((H1, 2)),
        ],
        out_shape=[
            jax.ShapeDtypeStruct((H1, B), jnp.bfloat16),
            jax.ShapeDtypeStruct((H1, 2), jnp.float32),
        ],
        scratch_shapes=[pltpu.VMEM((H1, 2), jnp.float32)],
        compiler_params=_seq,
    )(emb_t, x_num, W1e, W1n, b1c)

    h2, stats2 = pl.pallas_call(
        _mlp2_body,
        grid=(_NB // 2,),
        in_specs=[
            pl.BlockSpec((H1, 2 * _BB), lambda i: (0, i)),
            _full((H1, 2)),
            _full((H1, 1)),
            _full((H1, 1)),
            _full((H2, H1)),
            _full((H2, 1)),
        ],
        out_specs=[
            pl.BlockSpec((H2, 2 * _BB), lambda i: (0, i)),
            _full((H2, 2)),
        ],
        out_shape=[
            jax.ShapeDtypeStruct((H2, B), jnp.bfloat16),
            jax.ShapeDtypeStruct((H2, 2), jnp.float32),
        ],
        scratch_shapes=[pltpu.VMEM((H2, 2), jnp.float32)],
        compiler_params=_seq,
    )(h1, stats1, g1c, be1c, W2, b2c)

    out = pl.pallas_call(
        _mlp3_body,
        grid=(_NB // 2,),
        in_specs=[
            pl.BlockSpec((H2, 2 * _BB), lambda i: (0, i)),
            _full((H2, 2)),
            _full((H2, 1)),
            _full((H2, 1)),
            _full((H2, 1)),
            _full((1, 1)),
        ],
        out_specs=pl.BlockSpec((1, 2 * _BB), lambda i: (0, i)),
        out_shape=jax.ShapeDtypeStruct((1, B), jnp.float32),
        compiler_params=_seq,
    )(h2, stats2, g2c, be2c, w3c, b3c)

    return out.reshape(B)


# speedup vs baseline: 7.9632x; 1.0297x over previous
import functools

import jax
import jax.numpy as jnp
from jax import lax
from jax.experimental import pallas as pl
from jax.experimental.pallas import tpu as pltpu
from jax.experimental.pallas import tpu_sc as plsc

B = 16384
F = 26
V = 100000
E = 16
ND = 13
H1, H2 = 256, 128

_NC = 2
_NS = 16
_NW = _NC * _NS
_P = F * E
_PPW = _P // _NW
_BC = 4096
_NBC = B // _BC


def _build_gather():
    mesh = plsc.VectorSubcoreMesh(core_axis_name="c", subcore_axis_name="s")

    @functools.partial(
        pl.kernel,
        mesh=mesh,
        compiler_params=pltpu.CompilerParams(needs_layout_passes=False),
        out_type=jax.ShapeDtypeStruct((_P, B), jnp.float32),
        scratch_types=[
            pltpu.VMEM((V,), jnp.float32),
            pltpu.VMEM((B,), jnp.int32),
            pltpu.VMEM((_BC,), jnp.float32),
            pltpu.VMEM((_BC,), jnp.float32),
            pltpu.SemaphoreType.DMA,
            pltpu.SemaphoreType.DMA,
            pltpu.SemaphoreType.DMA,
            pltpu.SemaphoreType.DMA,
        ],
    )
    def gather_k(tt_hbm, idxt_hbm, out_hbm, plane_v, idxf, ob0, ob1,
                 psem, isem, osem0, osem1):
        wid = lax.axis_index("s") * _NC + lax.axis_index("c")
        p0 = wid * _PPW
        obs = (ob0, ob1)
        osems = (osem0, osem1)

        @pl.loop(0, _PPW)
        def _plane(p):
            pe = p0 + p
            f = pe // E
            e = pe % E
            pltpu.async_copy(tt_hbm.at[f, e], plane_v, psem)
            new_f = jnp.logical_or(p == 0, e == 0)

            @pl.when(new_f)
            def _():
                pltpu.async_copy(idxt_hbm.at[f], idxf, isem)

            @pl.when(p > 0)
            def _():
                for b in range(2):
                    pltpu.make_async_copy(
                        obs[b], out_hbm.at[pe, pl.ds(0, _BC)], osems[b]
                    ).wait()

            @pl.when(new_f)
            def _():
                pltpu.make_async_copy(idxt_hbm.at[f], idxf, isem).wait()

            pltpu.make_async_copy(tt_hbm.at[f, e], plane_v, psem).wait()

            @pl.loop(0, _NBC, step=2)
            def _c2(c0):
                for b in range(2):
                    c = c0 + b

                    @pl.when(c >= 2)
                    def _():
                        pltpu.make_async_copy(
                            obs[b], out_hbm.at[pe, pl.ds(0, _BC)], osems[b]
                        ).wait()

                    base = c * _BC

                    @plsc.parallel_loop(0, _BC // 16, unroll=8)
                    def _g(i):
                        ii = idxf[pl.ds(base + i * 16, 16)]
                        obs[b][pl.ds(i * 16, 16)] = plsc.load_gather(
                            plane_v, [ii])

                    pltpu.async_copy(
                        obs[b], out_hbm.at[pe, pl.ds(c * _BC, _BC)], osems[b])

        for b in range(2):
            pltpu.make_async_copy(
                obs[b], out_hbm.at[0, pl.ds(0, _BC)], osems[b]
            ).wait()

    return gather_k


_sc_gather = _build_gather()


_BB = 1024
_NB = B // _BB


def _gelu(x):
    return 0.5 * x * (1.0 + lax.erf(x * 0.7071067811865476))


def _mlp1_body(emb_ref, xn_ref, w1e_ref, w1n_ref, b1_ref, h1_ref, stats_ref,
               acc_ref):
    i = pl.program_id(0)

    @pl.when(i == 0)
    def _():
        acc_ref[...] = jnp.zeros_like(acc_ref)

    h = (lax.dot_general(w1e_ref[...], emb_ref[...], (((1,), (0,)), ((), ())),
                         preferred_element_type=jnp.float32)
         + lax.dot_general(w1n_ref[...], xn_ref[...], (((1,), (1,)), ((), ())),
                           preferred_element_type=jnp.float32)
         + b1_ref[...])
    h1_ref[...] = h.astype(jnp.bfloat16)
    acc_ref[...] += jnp.concatenate(
        [jnp.sum(h, axis=1, keepdims=True),
         jnp.sum(h * h, axis=1, keepdims=True)], axis=1)

    @pl.when(i == pl.num_programs(0) - 1)
    def _():
        stats_ref[...] = acc_ref[...]


def _mlp2_body(h1_ref, stats_ref, g1_ref, be1_ref, w2_ref, b2_ref, h2_ref,
               stats2_ref, acc_ref):
    i = pl.program_id(0)

    @pl.when(i == 0)
    def _():
        acc_ref[...] = jnp.zeros_like(acc_ref)

    mu = stats_ref[:, 0:1] * (1.0 / B)
    var = stats_ref[:, 1:2] * (1.0 / B) - mu * mu
    inv = lax.rsqrt(var + 1e-5)
    h1 = h1_ref[...].astype(jnp.float32)
    a = _gelu((h1 - mu) * (inv * g1_ref[...]) + be1_ref[...])
    h = (lax.dot_general(w2_ref[...], a, (((1,), (0,)), ((), ())),
                         preferred_element_type=jnp.float32) + b2_ref[...])
    h2_ref[...] = h.astype(jnp.bfloat16)
    acc_ref[...] += jnp.concatenate(
        [jnp.sum(h, axis=1, keepdims=True),
         jnp.sum(h * h, axis=1, keepdims=True)], axis=1)

    @pl.when(i == pl.num_programs(0) - 1)
    def _():
        stats2_ref[...] = acc_ref[...]


def _mlp3_body(h2_ref, stats2_ref, g2_ref, be2_ref, w3_ref, b3_ref, out_ref):
    mu = stats2_ref[:, 0:1] * (1.0 / B)
    var = stats2_ref[:, 1:2] * (1.0 / B) - mu * mu
    inv = lax.rsqrt(var + 1e-5)
    h2 = h2_ref[...].astype(jnp.float32)
    a = _gelu((h2 - mu) * (inv * g2_ref[...]) + be2_ref[...])
    o = jnp.sum(a * w3_ref[...], axis=0, keepdims=True)
    out_ref[...] = o + b3_ref[...]


def _full(shape):
    return pl.BlockSpec(shape, lambda i: (0, 0))


_seq = pltpu.CompilerParams(dimension_semantics=("arbitrary",))


def kernel(x_num, tables, W1, b1, g1, be1, W2, b2, g2, be2, W3, b3, x_cat):
    tt = jnp.swapaxes(tables, 1, 2)
    idx_t = x_cat.astype(jnp.int32).T
    emb_t = _sc_gather(tt, idx_t)

    W1e = W1[:, :F * E]
    W1n = W1[:, F * E:]
    b1c = b1.reshape(H1, 1)
    g1c = g1.reshape(H1, 1)
    be1c = be1.reshape(H1, 1)
    b2c = b2.reshape(H2, 1)
    g2c = g2.reshape(H2, 1)
    be2c = be2.reshape(H2, 1)
    w3c = W3.reshape(H2, 1)
    b3c = b3.reshape(1, 1)

    h1, stats1 = pl.pallas_call(
        _mlp1_body,
        grid=(_NB // 2,),
        in_specs=[
            pl.BlockSpec((_P, 2 * _BB), lambda i: (0, i)),
            pl.BlockSpec((2 * _BB, ND), lambda i: (i, 0)),
            _full((H1, _P)),
            _full((H1, ND)),
            _full((H1, 1)),
        ],
        out_specs=[
            pl.BlockSpec((H1, 2 * _BB), lambda i: (0, i)),
            _full((H1, 2)),
        ],
        out_shape=[
            jax.ShapeDtypeStruct((H1, B), jnp.bfloat16),
            jax.ShapeDtypeStruct((H1, 2), jnp.float32),
        ],
        scratch_shapes=[pltpu.VMEM((H1, 2), jnp.float32)],
        compiler_params=_seq,
    )(emb_t, x_num, W1e, W1n, b1c)

    h2, stats2 = pl.pallas_call(
        _mlp2_body,
        grid=(_NB // 2,),
        in_specs=[
            pl.BlockSpec((H1, 2 * _BB), lambda i: (0, i)),
            _full((H1, 2)),
            _full((H1, 1)),
            _full((H1, 1)),
            _full((H2, H1)),
            _full((H2, 1)),
        ],
        out_specs=[
            pl.BlockSpec((H2, 2 * _BB), lambda i: (0, i)),
            _full((H2, 2)),
        ],
        out_shape=[
            jax.ShapeDtypeStruct((H2, B), jnp.bfloat16),
            jax.ShapeDtypeStruct((H2, 2), jnp.float32),
        ],
        scratch_shapes=[pltpu.VMEM((H2, 2), jnp.float32)],
        compiler_params=_seq,
    )(h1, stats1, g1c, be1c, W2, b2c)

    out = pl.pallas_call(
        _mlp3_body,
        grid=(_NB // 2,),
        in_specs=[
            pl.BlockSpec((H2, 2 * _BB), lambda i: (0, i)),
            _full((H2, 2)),
            _full((H2, 1)),
            _full((H2, 1)),
            _full((H2, 1)),
            _full((1, 1)),
        ],
        out_specs=pl.BlockSpec((1, 2 * _BB), lambda i: (0, i)),
        out_shape=jax.ShapeDtypeStruct((1, B), jnp.float32),
        compiler_params=_seq,
    )(h2, stats2, g2c, be2c, w3c, b3c)

    return out.reshape(B)
